# baseline (speedup 1.0000x reference)
import functools

import jax
import jax.numpy as jnp
from jax import lax
from jax.experimental import pallas as pl
from jax.experimental.pallas import tpu as pltpu
from jax.experimental.pallas import tpu_sc as plsc

_RADIUS2 = 0.2 * 0.2
_NSAMPLE = 32
_P = 64
_NC = 512


def _ballq_body(xyzT_ref, nqT_ref, gxyz_ref, idx_ref):
    b = pl.program_id(0)
    n = xyzT_ref.shape[2]
    xyzT = xyzT_ref[0]
    nqT = nqT_ref[0, 0]

    qq = jnp.sum(nqT * nqT, axis=0)

    ii = lax.broadcasted_iota(jnp.int32, (_NC, _NC), 0)
    jj = lax.broadcasted_iota(jnp.int32, (_NC, _NC), 1)
    tri = (ii <= jj).astype(jnp.float32)

    s_keys3 = (2.0 * lax.broadcasted_iota(
        jnp.int32, (_P, _NSAMPLE, _NC), 1).astype(jnp.float32) + 1.0)

    carry = jnp.zeros((_P, 1), jnp.float32)
    acc = jnp.zeros((4, _P * _NSAMPLE), jnp.float32)

    def chunk_body(c, acc, carry):
        xc_full = xyzT[:, c * _NC:(c + 1) * _NC]
        xx_c = jnp.sum(xc_full * xc_full, axis=0)
        cross_c = lax.dot_general(nqT, xc_full, (((0,), (0,)), ((), ())),
                                  preferred_element_type=jnp.float32)
        dist2_c = (qq[:, None] + xx_c[None, :]) - 2.0 * cross_c
        mask_c = (dist2_c <= _RADIUS2).astype(jnp.float32)
        rank_c = lax.dot_general(mask_c, tri, (((1,), (0,)), ((), ())),
                                 preferred_element_type=jnp.float32,
                                 precision=lax.Precision.HIGHEST)
        rank_c = rank_c + carry
        key_c = 2.0 * rank_c - mask_c
        w3 = (key_c[:, None, :] == s_keys3).astype(jnp.float32)
        w2 = w3.reshape(_P * _NSAMPLE, _NC)
        xc = xc_full
        nrow = (lax.broadcasted_iota(jnp.int32, (1, _NC), 1)
                .astype(jnp.float32) + float(c * _NC))
        x4 = jnp.concatenate([xc, nrow], axis=0)
        acc = acc + lax.dot_general(x4, w2, (((1,), (1,)), ((), ())),
                                    preferred_element_type=jnp.float32,
                                    precision=lax.Precision.HIGHEST)
        carry = rank_c[:, _NC - 1:_NC]
        return acc, carry

    for c in range(n // _NC):
        if c == 0:
            acc, carry = chunk_body(0, acc, carry)
        else:
            acc, carry = lax.cond(
                jnp.min(carry) < float(_NSAMPLE),
                lambda a, k, c=c: chunk_body(c, a, k),
                lambda a, k: (a, k),
                acc, carry)

    ps = _P * _NSAMPLE
    qi = lax.broadcasted_iota(jnp.int32, (_P, ps), 0)
    li = lax.broadcasted_iota(jnp.int32, (_P, ps), 1)
    e = (qi == li // _NSAMPLE).astype(jnp.float32)
    ri = lax.broadcasted_iota(jnp.int32, (ps, _P), 0)
    ci = lax.broadcasted_iota(jnp.int32, (ps, _P), 1)
    s0 = (ri == ci * _NSAMPLE).astype(jnp.float32)
    s2048 = (lax.broadcasted_iota(jnp.int32, (1, ps), 1)
             % _NSAMPLE).astype(jnp.float32)

    k2048 = lax.dot_general(carry, e, (((0,), (0,)), ((), ())),
                            preferred_element_type=jnp.float32,
                            precision=lax.Precision.HIGHEST)
    valid = s2048 < k2048
    has = k2048 > 0.0

    acc3 = acc[3:4]
    firstq = jnp.dot(acc3, s0, preferred_element_type=jnp.float32, precision=lax.Precision.HIGHEST)
    first2048 = jnp.dot(firstq, e, preferred_element_type=jnp.float32, precision=lax.Precision.HIGHEST)
    idx_fix = jnp.where(valid, acc3,
                        jnp.where(has, first2048, float(n - 1)))
    idx_ref[0, 0] = idx_fix.astype(jnp.int32) + b * n

    g3 = acc[0:3]
    firstgq = jnp.dot(g3, s0, preferred_element_type=jnp.float32, precision=lax.Precision.HIGHEST)
    firstg = jnp.dot(firstgq, e, preferred_element_type=jnp.float32, precision=lax.Precision.HIGHEST)
    lastx = jnp.broadcast_to(xyzT[:, n - 1:n], (3, ps))
    g_fix = jnp.where(valid, g3, jnp.where(has, firstg, lastx))
    nq2048 = jnp.dot(nqT, e, preferred_element_type=jnp.float32, precision=lax.Precision.HIGHEST)
    gxyz_ref[0, 0] = g_fix - nq2048


def _ball_query_group_xyz(xyzT, nqT):
    b, _, n = xyzT.shape
    npoint = nqT.shape[2]
    nblk = npoint // _P
    nq4 = nqT.reshape(b, 3, nblk, _P).transpose(0, 2, 1, 3)
    grid = (b, nblk)
    return pl.pallas_call(
        _ballq_body,
        grid=grid,
        in_specs=[
            pl.BlockSpec((1, 3, n), lambda i, j: (i, 0, 0)),
            pl.BlockSpec((1, 1, 3, _P), lambda i, j: (i, j, 0, 0)),
        ],
        out_specs=[
            pl.BlockSpec((1, 1, 3, _P * _NSAMPLE), lambda i, j: (i, j, 0, 0)),
            pl.BlockSpec((1, 1, 1, _P * _NSAMPLE), lambda i, j: (i, j, 0, 0)),
        ],
        out_shape=[
            jax.ShapeDtypeStruct((b, nblk, 3, _P * _NSAMPLE), jnp.float32),
            jax.ShapeDtypeStruct((b, nblk, 1, _P * _NSAMPLE), jnp.int32),
        ],
    )(xyzT, nq4)


_GATH = 128


def _sc_gather(table, idx):
    total = idx.shape[0]
    c = table.shape[1]
    info = plsc.get_sparse_core_info()
    nw = info.num_cores * info.num_subcores
    per_w = total // nw
    iters = per_w // _GATH
    mesh = plsc.VectorSubcoreMesh(core_axis_name="c", subcore_axis_name="s")

    @functools.partial(
        pl.kernel,
        mesh=mesh,
        out_type=jax.ShapeDtypeStruct((total, c), jnp.float32),
        scratch_types=[
            pltpu.VMEM((_GATH,), jnp.int32),
            pltpu.VMEM((_GATH, c), jnp.float32),
            pltpu.SemaphoreType.DMA,
        ],
    )
    def k(table_hbm, idx_hbm, out_hbm, idx_v, rows_v, sem):
        wid = lax.axis_index("s") * info.num_cores + lax.axis_index("c")
        base = wid * per_w

        def body(i, _):
            off = base + i * _GATH
            pltpu.sync_copy(idx_hbm.at[pl.ds(off, _GATH)], idx_v)
            pltpu.async_copy(table_hbm.at[idx_v], rows_v, sem).wait()
            pltpu.sync_copy(rows_v, out_hbm.at[pl.ds(off, _GATH)])
            return 0

        lax.fori_loop(0, iters, body, 0)

    return k(table, idx)


def kernel(xyz, new_xyz, features):
    b, n, _ = xyz.shape
    npoint = new_xyz.shape[1]
    c = features.shape[1]

    xyzT = jnp.transpose(xyz, (0, 2, 1))
    nqT = jnp.transpose(new_xyz, (0, 2, 1))

    gxyz_blk, idx_blk = _ball_query_group_xyz(xyzT, nqT)
    nblk = npoint // _P
    gxyz = (gxyz_blk.reshape(b, nblk, 3, _P, _NSAMPLE)
            .transpose(0, 2, 1, 3, 4).reshape(b, 3, npoint, _NSAMPLE))

    featT = jnp.transpose(features, (0, 2, 1)).reshape(b * n, c)
    rows = _sc_gather(featT, idx_blk.reshape(-1))

    gfeat = rows.reshape(b, npoint, _NSAMPLE, c).transpose(0, 3, 1, 2)
    return jnp.concatenate([gxyz, gfeat], axis=1)


# speedup vs baseline: 1.0404x; 1.0404x over previous
import functools

import jax
import jax.numpy as jnp
from jax import lax
from jax.experimental import pallas as pl
from jax.experimental.pallas import tpu as pltpu
from jax.experimental.pallas import tpu_sc as plsc

_RADIUS2 = 0.2 * 0.2
_NSAMPLE = 32
_P = 64
_NC = 512


def _ballq_body(xyzT_ref, nqT_ref, gxyz_ref, idx_ref):
    b = pl.program_id(0)
    n = xyzT_ref.shape[2]
    xyzT = xyzT_ref[0]
    nqT = nqT_ref[0, 0]

    xx = jnp.sum(xyzT * xyzT, axis=0)
    qq = jnp.sum(nqT * nqT, axis=0)
    cross = lax.dot_general(nqT, xyzT, (((0,), (0,)), ((), ())),
                            preferred_element_type=jnp.float32)
    dist2 = (qq[:, None] + xx[None, :]) - 2.0 * cross
    mask = (dist2 <= _RADIUS2).astype(jnp.float32)

    ii = lax.broadcasted_iota(jnp.int32, (_NC, _NC), 0)
    jj = lax.broadcasted_iota(jnp.int32, (_NC, _NC), 1)
    tri = (ii <= jj).astype(jnp.float32)

    s_keys3 = (2.0 * lax.broadcasted_iota(
        jnp.int32, (_P, _NSAMPLE, _NC), 1).astype(jnp.float32) + 1.0)

    carry = jnp.zeros((_P, 1), jnp.float32)
    acc = jnp.zeros((4, _P * _NSAMPLE), jnp.float32)

    def chunk_body(c, acc, carry):
        xc_full = xyzT[:, c * _NC:(c + 1) * _NC]
        mask_c = mask[:, c * _NC:(c + 1) * _NC]
        rank_c = lax.dot_general(mask_c, tri, (((1,), (0,)), ((), ())),
                                 preferred_element_type=jnp.float32,
                                 precision=lax.Precision.HIGHEST)
        rank_c = rank_c + carry
        key_c = 2.0 * rank_c - mask_c
        w3 = (key_c[:, None, :] == s_keys3).astype(jnp.float32)
        w2 = w3.reshape(_P * _NSAMPLE, _NC)
        xc = xc_full
        nrow = (lax.broadcasted_iota(jnp.int32, (1, _NC), 1)
                .astype(jnp.float32) + float(c * _NC))
        x4 = jnp.concatenate([xc, nrow], axis=0)
        acc = acc + lax.dot_general(x4, w2, (((1,), (1,)), ((), ())),
                                    preferred_element_type=jnp.float32,
                                    precision=lax.Precision.HIGHEST)
        carry = rank_c[:, _NC - 1:_NC]
        return acc, carry

    for c in range(n // _NC):
        if c == 0:
            acc, carry = chunk_body(0, acc, carry)
        else:
            acc, carry = lax.cond(
                jnp.min(carry) < float(_NSAMPLE),
                lambda a, k, c=c: chunk_body(c, a, k),
                lambda a, k: (a, k),
                acc, carry)

    ps = _P * _NSAMPLE
    qi = lax.broadcasted_iota(jnp.int32, (_P, ps), 0)
    li = lax.broadcasted_iota(jnp.int32, (_P, ps), 1)
    e = (qi == li // _NSAMPLE).astype(jnp.float32)
    ri = lax.broadcasted_iota(jnp.int32, (ps, _P), 0)
    ci = lax.broadcasted_iota(jnp.int32, (ps, _P), 1)
    s0 = (ri == ci * _NSAMPLE).astype(jnp.float32)
    s2048 = (lax.broadcasted_iota(jnp.int32, (1, ps), 1)
             % _NSAMPLE).astype(jnp.float32)

    ident = (lax.broadcasted_iota(jnp.int32, (_P, _P), 0)
             == lax.broadcasted_iota(jnp.int32, (_P, _P), 1)).astype(jnp.float32)
    firstq4 = jnp.dot(acc, s0, preferred_element_type=jnp.float32,
                      precision=lax.Precision.HIGHEST)
    carryT = lax.dot_general(carry, ident, (((0,), (0,)), ((), ())),
                             preferred_element_type=jnp.float32,
                             precision=lax.Precision.HIGHEST)
    packed = jnp.concatenate([firstq4, carryT, nqT], axis=0)
    exp = jnp.dot(packed, e, preferred_element_type=jnp.float32,
                  precision=lax.Precision.HIGHEST)
    firstg = exp[0:3]
    first2048 = exp[3:4]
    k2048 = exp[4:5]
    nq2048 = exp[5:8]
    valid = s2048 < k2048
    has = k2048 > 0.0

    acc3 = acc[3:4]
    idx_fix = jnp.where(valid, acc3,
                        jnp.where(has, first2048, float(n - 1)))
    idx_ref[0, 0] = idx_fix.astype(jnp.int32) + b * n

    g3 = acc[0:3]
    lastx = jnp.broadcast_to(xyzT[:, n - 1:n], (3, ps))
    g_fix = jnp.where(valid, g3, jnp.where(has, firstg, lastx))
    gxyz_ref[0, 0] = g_fix - nq2048


def _ball_query_group_xyz(xyzT, nqT):
    b, _, n = xyzT.shape
    npoint = nqT.shape[2]
    nblk = npoint // _P
    nq4 = nqT.reshape(b, 3, nblk, _P).transpose(0, 2, 1, 3)
    grid = (b, nblk)
    return pl.pallas_call(
        _ballq_body,
        grid=grid,
        in_specs=[
            pl.BlockSpec((1, 3, n), lambda i, j: (i, 0, 0)),
            pl.BlockSpec((1, 1, 3, _P), lambda i, j: (i, j, 0, 0)),
        ],
        out_specs=[
            pl.BlockSpec((1, 1, 3, _P * _NSAMPLE), lambda i, j: (i, j, 0, 0)),
            pl.BlockSpec((1, 1, 1, _P * _NSAMPLE), lambda i, j: (i, j, 0, 0)),
        ],
        out_shape=[
            jax.ShapeDtypeStruct((b, nblk, 3, _P * _NSAMPLE), jnp.float32),
            jax.ShapeDtypeStruct((b, nblk, 1, _P * _NSAMPLE), jnp.int32),
        ],
    )(xyzT, nq4)


_GATH = 128


def _sc_gather(table, idx):
    total = idx.shape[0]
    c = table.shape[1]
    info = plsc.get_sparse_core_info()
    nw = info.num_cores * info.num_subcores
    per_w = total // nw
    iters = per_w // _GATH
    mesh = plsc.VectorSubcoreMesh(core_axis_name="c", subcore_axis_name="s")

    @functools.partial(
        pl.kernel,
        mesh=mesh,
        out_type=jax.ShapeDtypeStruct((total, c), jnp.float32),
        scratch_types=[
            pltpu.VMEM((_GATH,), jnp.int32),
            pltpu.VMEM((_GATH, c), jnp.float32),
            pltpu.SemaphoreType.DMA,
        ],
    )
    def k(table_hbm, idx_hbm, out_hbm, idx_v, rows_v, sem):
        wid = lax.axis_index("s") * info.num_cores + lax.axis_index("c")
        base = wid * per_w

        def body(i, _):
            off = base + i * _GATH
            pltpu.sync_copy(idx_hbm.at[pl.ds(off, _GATH)], idx_v)
            pltpu.async_copy(table_hbm.at[idx_v], rows_v, sem).wait()
            pltpu.sync_copy(rows_v, out_hbm.at[pl.ds(off, _GATH)])
            return 0

        lax.fori_loop(0, iters, body, 0)

    return k(table, idx)


def kernel(xyz, new_xyz, features):
    b, n, _ = xyz.shape
    npoint = new_xyz.shape[1]
    c = features.shape[1]

    xyzT = jnp.transpose(xyz, (0, 2, 1))
    nqT = jnp.transpose(new_xyz, (0, 2, 1))

    gxyz_blk, idx_blk = _ball_query_group_xyz(xyzT, nqT)
    nblk = npoint // _P
    gxyz = (gxyz_blk.reshape(b, nblk, 3, _P, _NSAMPLE)
            .transpose(0, 2, 1, 3, 4).reshape(b, 3, npoint, _NSAMPLE))

    featT = jnp.transpose(features, (0, 2, 1)).reshape(b * n, c)
    rows = _sc_gather(featT, idx_blk.reshape(-1))

    gfeat = rows.reshape(b, npoint, _NSAMPLE, c).transpose(0, 3, 1, 2)
    return jnp.concatenate([gxyz, gfeat], axis=1)
